# Initial kernel scaffold; baseline (speedup 1.0000x reference)
import jax
import jax.numpy as jnp
from jax.experimental import pallas as pl


def kernel(x, alpha):
    raise NotImplementedError("write your pallas kernel here")


# speedup vs baseline: 213.8013x; 213.8013x over previous
import functools

import jax
import jax.numpy as jnp
from jax import lax
from jax.experimental import pallas as pl
from jax.experimental.pallas import tpu as pltpu

D = 9
R = 4
YR = 256
XR = 256
H = YR * D
W = XR * D
BR = 48
NB = H // BR


def _tap_weights(idx_f32, idx_i32, alpha, n_res, off):
    v = idx_i32 % D
    y = idx_i32 // D
    t = -alpha * (v - R).astype(jnp.float32)
    m = jnp.floor(t)
    f = t - m
    mi = m.astype(jnp.int32)
    j1 = jnp.clip(y + mi, 0, n_res - 1) - y
    j2 = jnp.clip(y + mi + 1, 0, n_res - 1) - y
    return jnp.where(j1 == off, 1.0 - f, 0.0) + jnp.where(j2 == off, f, 0.0)


def _body(alpha_ref, prev_ref, cur_ref, next_ref, out_ref, win_ref, pad_ref):
    rb = pl.program_id(1)
    alpha = alpha_ref[0]

    win_ref[0:BR, :] = prev_ref[0, 0]
    win_ref[BR:2 * BR, :] = cur_ref[0, 0]
    win_ref[2 * BR:3 * BR, :] = next_ref[0, 0]

    row = lax.broadcasted_iota(jnp.int32, (BR, 1), 0) + rb * BR
    col = lax.broadcasted_iota(jnp.int32, (1, W), 1)

    tmp = jnp.zeros((BR, W), jnp.float32)
    for off in range(-4, 6):
        b = _tap_weights(None, row, alpha, YR, off)
        tmp = tmp + b * win_ref[BR + D * off:BR + D * off + BR, :]

    pad_ref[:, 0:BR] = jnp.zeros((BR, BR), jnp.float32)
    pad_ref[:, BR:BR + W] = tmp
    pad_ref[:, BR + W:] = jnp.zeros((BR, BR), jnp.float32)

    out = jnp.zeros((BR, W), jnp.float32)
    for off in range(-4, 6):
        a = _tap_weights(None, col, alpha, XR, off)
        out = out + a * pad_ref[:, BR + D * off:BR + D * off + W]
    out_ref[0, 0] = out


@jax.jit
def kernel(x, alpha):
    alpha_arr = jnp.reshape(alpha.astype(jnp.float32), (1,))

    grid = (3, NB)
    blk = (1, 1, BR, W)

    def im_prev(c, rb):
        return (0, c, jnp.maximum(rb - 1, 0), 0)

    def im_cur(c, rb):
        return (0, c, rb, 0)

    def im_next(c, rb):
        return (0, c, jnp.minimum(rb + 1, NB - 1), 0)

    out = pl.pallas_call(
        _body,
        grid=grid,
        in_specs=[
            pl.BlockSpec(memory_space=pltpu.SMEM),
            pl.BlockSpec(blk, im_prev),
            pl.BlockSpec(blk, im_cur),
            pl.BlockSpec(blk, im_next),
        ],
        out_specs=pl.BlockSpec(blk, im_cur),
        out_shape=jax.ShapeDtypeStruct((1, 3, H, W), jnp.float32),
        scratch_shapes=[
            pltpu.VMEM((3 * BR, W), jnp.float32),
            pltpu.VMEM((BR, W + 2 * BR), jnp.float32),
        ],
    )(alpha_arr, x, x, x)
    return out
